# baseline (speedup 1.0000x reference)
import functools

import jax
import jax.numpy as jnp
from jax import lax
from jax.experimental import pallas as pl
from jax.experimental.pallas import tpu as pltpu
from jax.experimental.pallas import tpu_sc as plsc

AUXW = 0.01
_NC, _NS, _L = 2, 16, 16


def _logits_body(x_ref, rw_ref, out_ref):
    out_ref[...] = jax.lax.dot_general(
        x_ref[...], rw_ref[...], (((1,), (1,)), ((), ())),
        preferred_element_type=jnp.float32)


def _make_route_sc(n, e):
    nw = _NC * _NS
    tpw = n // nw
    nch = tpw // _L
    mesh = plsc.VectorSubcoreMesh(core_axis_name="c", subcore_axis_name="s")

    @functools.partial(
        pl.kernel, mesh=mesh,
        out_type=(
            jax.ShapeDtypeStruct((n * e,), jnp.float32),
            jax.ShapeDtypeStruct((nw, _L), jnp.float32),
        ),
        compiler_params=pltpu.CompilerParams(needs_layout_passes=False),
        scratch_types=[
            pltpu.VMEM((tpw * e,), jnp.float32),
            pltpu.VMEM((tpw * e,), jnp.float32),
            pltpu.VMEM((_L,), jnp.float32),
        ],
    )
    def route(logits_hbm, wf_hbm, sall_hbm, lv, wfl, svv):
        wid = lax.axis_index("s") * _NC + lax.axis_index("c")
        base = wid * tpw
        pltpu.sync_copy(logits_hbm.at[pl.ds(base * e, tpw * e)], lv)
        lane = lax.broadcasted_iota(jnp.int32, (_L,), 0)
        accs = [jnp.zeros((_L,), jnp.float32) for _ in range(e)]
        for c in range(nch):
            rows = lane + (c * _L)
            vs = [plsc.load_gather(lv, [rows * e + ei]) for ei in range(e)]
            m = vs[0]
            for v in vs[1:]:
                m = jnp.maximum(m, v)
            exs = [jnp.exp(v - m) for v in vs]
            ssum = exs[0]
            for ex in exs[1:]:
                ssum = ssum + ex
            ps = [ex / ssum for ex in exs]
            m1 = ps[0]
            for p in ps[1:]:
                m1 = jnp.maximum(m1, p)
            i1 = jnp.full((_L,), e, jnp.int32)
            for ei in range(e):
                i1 = jnp.minimum(i1, jnp.where(
                    ps[ei] == m1, jnp.full((_L,), ei, jnp.int32),
                    jnp.full((_L,), e, jnp.int32)))
            pm = [jnp.where(jnp.full((_L,), ei, jnp.int32) == i1,
                            jnp.full((_L,), -1.0, jnp.float32), ps[ei])
                  for ei in range(e)]
            m2 = pm[0]
            for p in pm[1:]:
                m2 = jnp.maximum(m2, p)
            i2 = jnp.full((_L,), e, jnp.int32)
            for ei in range(e):
                i2 = jnp.minimum(i2, jnp.where(
                    pm[ei] == m2, jnp.full((_L,), ei, jnp.int32),
                    jnp.full((_L,), e, jnp.int32)))
            denom = m1 + m2
            w1 = m1 / denom
            w2 = m2 / denom
            for ei in range(e):
                eivec = jnp.full((_L,), ei, jnp.int32)
                wfe = jnp.where(eivec == i1, w1,
                                jnp.where(eivec == i2, w2,
                                          jnp.zeros((_L,), jnp.float32)))
                plsc.store_scatter(wfl, [rows * e + ei], wfe)
                accs[ei] = accs[ei] + wfe
        sv = jnp.zeros((_L,), jnp.float32)
        for ei in range(e):
            sv = jnp.where(lane == ei, jnp.sum(accs[ei]), sv)
        svv[...] = sv
        pltpu.sync_copy(wfl, wf_hbm.at[pl.ds(base * e, tpw * e)])
        pltpu.sync_copy(svv, sall_hbm.at[wid])

    return route


def _ffn_body(eo_ref, na_ref, x_ref, wf_ref, gw_ref, gb_ref, uw_ref, ub_ref,
              dw_ref, db_ref, out_ref, wcol_ref, *, n, e, nib):
    ei = pl.program_id(0)
    ii = pl.program_id(1)

    @pl.when((ei == 0) & (ii == 0))
    def _():
        out_ref[...] = jnp.zeros_like(out_ref)

    @pl.when(ei < na_ref[0])
    def _():
        emap = eo_ref[jnp.minimum(ei, na_ref[0] - 1)]

        @pl.when(ii == 0)
        def _():
            onehot = (jax.lax.broadcasted_iota(jnp.int32, (e, 1), 0)
                      == emap).astype(jnp.float32)
            wcol_ref[...] = jax.lax.dot_general(
                wf_ref[...], onehot, (((1,), (0,)), ((), ())),
                preferred_element_type=jnp.float32)
            out_ref[...] = out_ref[...] + wcol_ref[...] * db_ref[0]

        wcol = wcol_ref[...]
        x = x_ref[...]
        gw = gw_ref[0].astype(jnp.bfloat16)
        uw = uw_ref[0].astype(jnp.bfloat16)
        dw = dw_ref[0].astype(jnp.bfloat16)
        g = jax.lax.dot_general(x, gw, (((1,), (1,)), ((), ())),
                                preferred_element_type=jnp.float32)
        g = g + gb_ref[0, 0]
        u = jax.lax.dot_general(x, uw, (((1,), (1,)), ((), ())),
                                preferred_element_type=jnp.float32)
        u = u + ub_ref[0, 0]
        a = (g * jax.nn.sigmoid(g) * (u * wcol)).astype(jnp.bfloat16)
        part = jax.lax.dot_general(a, dw, (((1,), (1,)), ((), ())),
                                   preferred_element_type=jnp.float32)
        out_ref[...] = out_ref[...] + part


def kernel(hidden_states, router_W, gate_W, gate_b, up_W, up_b, down_W,
           down_b):
    b, s, h = hidden_states.shape
    e, i_dim = gate_W.shape[:2]
    n = b * s
    x = hidden_states.reshape(n, h)

    logits = pl.pallas_call(
        _logits_body,
        out_shape=jax.ShapeDtypeStruct((n, e), jnp.float32),
    )(x, router_W)

    wff, sall = _make_route_sc(n, e)(logits.reshape(n * e))
    wf = wff.reshape(n, e)

    scol = jnp.sum(sall[:, :e], axis=0)
    aux = jnp.sum(scol * scol) * (AUXW / n)
    iota8 = jnp.arange(e, dtype=jnp.int32)
    active = scol > 0.0
    eorder = jnp.argsort(jnp.where(active, iota8, iota8 + e)).astype(jnp.int32)
    nact = jnp.sum(active.astype(jnp.int32)).reshape(1)

    ib = 512
    nib = i_dim // ib
    xb = x.astype(jnp.bfloat16)

    def wspec_in(ei, ii, eo, na):
        act = ei < na[0]
        eix = eo[jnp.where(act, ei, na[0] - 1)]
        iix = jnp.where(act, ii, nib - 1)
        return eix, iix

    grid_spec = pltpu.PrefetchScalarGridSpec(
        num_scalar_prefetch=2,
        grid=(e, nib),
        in_specs=[
            pl.BlockSpec((n, h), lambda ei, ii, eo, na: (0, 0)),
            pl.BlockSpec((n, e), lambda ei, ii, eo, na: (0, 0)),
            pl.BlockSpec((1, ib, h),
                         lambda ei, ii, eo, na: (*wspec_in(ei, ii, eo, na), 0)),
            pl.BlockSpec((1, 1, 1, ib),
                         lambda ei, ii, eo, na: (*wspec_in(ei, ii, eo, na), 0, 0)),
            pl.BlockSpec((1, ib, h),
                         lambda ei, ii, eo, na: (*wspec_in(ei, ii, eo, na), 0)),
            pl.BlockSpec((1, 1, 1, ib),
                         lambda ei, ii, eo, na: (*wspec_in(ei, ii, eo, na), 0, 0)),
            pl.BlockSpec((1, h, ib),
                         lambda ei, ii, eo, na:
                         (wspec_in(ei, ii, eo, na)[0], 0,
                          wspec_in(ei, ii, eo, na)[1])),
            pl.BlockSpec((1, 1, h),
                         lambda ei, ii, eo, na:
                         (wspec_in(ei, ii, eo, na)[0], 0, 0)),
        ],
        out_specs=pl.BlockSpec((n, h), lambda ei, ii, eo, na: (0, 0)),
        scratch_shapes=[pltpu.VMEM((n, 1), jnp.float32)],
    )

    combined = pl.pallas_call(
        functools.partial(_ffn_body, n=n, e=e, nib=nib),
        grid_spec=grid_spec,
        out_shape=jax.ShapeDtypeStruct((n, h), jnp.float32),
        compiler_params=pltpu.CompilerParams(
            dimension_semantics=("arbitrary", "arbitrary")),
    )(eorder, nact, xb, wf, gate_W, gate_b.reshape(e, nib, 1, ib), up_W,
      up_b.reshape(e, nib, 1, ib), down_W, down_b.reshape(e, 1, h))

    return combined.reshape(b, s, h), aux


# speedup vs baseline: 1.0055x; 1.0055x over previous
import functools

import jax
import jax.numpy as jnp
from jax import lax
from jax.experimental import pallas as pl
from jax.experimental.pallas import tpu as pltpu
from jax.experimental.pallas import tpu_sc as plsc

AUXW = 0.01
_NC, _NS, _L = 2, 16, 16


def _logits_body(x_ref, rw_ref, out_ref):
    out_ref[...] = jax.lax.dot_general(
        x_ref[...], rw_ref[...], (((1,), (1,)), ((), ())),
        preferred_element_type=jnp.float32)


def _make_route_sc(n, e):
    nw = _NC * _NS
    tpw = n // nw
    nch = tpw // _L
    mesh = plsc.VectorSubcoreMesh(core_axis_name="c", subcore_axis_name="s")

    @functools.partial(
        pl.kernel, mesh=mesh,
        out_type=(
            jax.ShapeDtypeStruct((n * e,), jnp.float32),
            jax.ShapeDtypeStruct((nw, _L), jnp.float32),
        ),
        compiler_params=pltpu.CompilerParams(needs_layout_passes=False),
        scratch_types=[
            pltpu.VMEM((tpw * e,), jnp.float32),
            pltpu.VMEM((tpw * e,), jnp.float32),
            pltpu.VMEM((_L,), jnp.float32),
        ],
    )
    def route(logits_hbm, wf_hbm, sall_hbm, lv, wfl, svv):
        wid = lax.axis_index("s") * _NC + lax.axis_index("c")
        base = wid * tpw
        pltpu.sync_copy(logits_hbm.at[pl.ds(base * e, tpw * e)], lv)
        lane = lax.broadcasted_iota(jnp.int32, (_L,), 0)
        accs = [jnp.zeros((_L,), jnp.float32) for _ in range(e)]
        for c in range(nch):
            rows = lane + (c * _L)
            vs = [plsc.load_gather(lv, [rows * e + ei]) for ei in range(e)]
            m = vs[0]
            for v in vs[1:]:
                m = jnp.maximum(m, v)
            exs = [jnp.exp(v - m) for v in vs]
            ssum = exs[0]
            for ex in exs[1:]:
                ssum = ssum + ex
            ps = [ex / ssum for ex in exs]
            m1 = ps[0]
            for p in ps[1:]:
                m1 = jnp.maximum(m1, p)
            i1 = jnp.full((_L,), e, jnp.int32)
            for ei in range(e):
                i1 = jnp.minimum(i1, jnp.where(
                    ps[ei] == m1, jnp.full((_L,), ei, jnp.int32),
                    jnp.full((_L,), e, jnp.int32)))
            pm = [jnp.where(jnp.full((_L,), ei, jnp.int32) == i1,
                            jnp.full((_L,), -1.0, jnp.float32), ps[ei])
                  for ei in range(e)]
            m2 = pm[0]
            for p in pm[1:]:
                m2 = jnp.maximum(m2, p)
            i2 = jnp.full((_L,), e, jnp.int32)
            for ei in range(e):
                i2 = jnp.minimum(i2, jnp.where(
                    pm[ei] == m2, jnp.full((_L,), ei, jnp.int32),
                    jnp.full((_L,), e, jnp.int32)))
            denom = m1 + m2
            w1 = m1 / denom
            w2 = m2 / denom
            for ei in range(e):
                eivec = jnp.full((_L,), ei, jnp.int32)
                wfe = jnp.where(eivec == i1, w1,
                                jnp.where(eivec == i2, w2,
                                          jnp.zeros((_L,), jnp.float32)))
                plsc.store_scatter(wfl, [rows * e + ei], wfe)
                accs[ei] = accs[ei] + wfe
        sv = jnp.zeros((_L,), jnp.float32)
        for ei in range(e):
            sv = jnp.where(lane == ei, jnp.sum(accs[ei]), sv)
        svv[...] = sv
        pltpu.sync_copy(wfl, wf_hbm.at[pl.ds(base * e, tpw * e)])
        pltpu.sync_copy(svv, sall_hbm.at[wid])

    return route


def _ffn_body(eo_ref, na_ref, x_ref, wf_ref, gw_ref, gb_ref, uw_ref, ub_ref,
              dw_ref, db_ref, out_ref, wcol_ref, *, n, e, nib):
    ei = pl.program_id(0)
    ii = pl.program_id(1)

    @pl.when((ei == 0) & (ii == 0))
    def _():
        out_ref[...] = jnp.zeros_like(out_ref)

    @pl.when(ei < na_ref[0])
    def _():
        emap = eo_ref[jnp.minimum(ei, na_ref[0] - 1)]

        @pl.when(ii == 0)
        def _():
            onehot = (jax.lax.broadcasted_iota(jnp.int32, (e, 1), 0)
                      == emap).astype(jnp.float32)
            wcol_ref[...] = jax.lax.dot_general(
                wf_ref[...], onehot, (((1,), (0,)), ((), ())),
                preferred_element_type=jnp.float32)
            out_ref[...] = out_ref[...] + wcol_ref[...] * db_ref[0]

        wcol = wcol_ref[...]
        x = x_ref[...]
        gw = gw_ref[0].astype(jnp.bfloat16)
        uw = uw_ref[0].astype(jnp.bfloat16)
        dw = dw_ref[0].astype(jnp.bfloat16)
        g = jax.lax.dot_general(x, gw, (((1,), (1,)), ((), ())),
                                preferred_element_type=jnp.float32)
        g = g + gb_ref[0, 0]
        u = jax.lax.dot_general(x, uw, (((1,), (1,)), ((), ())),
                                preferred_element_type=jnp.float32)
        u = u + ub_ref[0, 0]
        a = (g * jax.nn.sigmoid(g) * (u * wcol)).astype(jnp.bfloat16)
        part = jax.lax.dot_general(a, dw, (((1,), (1,)), ((), ())),
                                   preferred_element_type=jnp.float32)
        out_ref[...] = out_ref[...] + part


def kernel(hidden_states, router_W, gate_W, gate_b, up_W, up_b, down_W,
           down_b):
    b, s, h = hidden_states.shape
    e, i_dim = gate_W.shape[:2]
    n = b * s
    x = hidden_states.reshape(n, h)

    logits = pl.pallas_call(
        _logits_body,
        out_shape=jax.ShapeDtypeStruct((n, e), jnp.float32),
    )(x, router_W)

    wff, sall = _make_route_sc(n, e)(logits.reshape(n * e))
    wf = wff.reshape(n, e)

    scol = jnp.sum(sall[:, :e], axis=0)
    aux = jnp.sum(scol * scol) * (AUXW / n)
    iota8 = jnp.arange(e, dtype=jnp.int32)
    active = scol > 0.0
    eorder = jnp.argsort(jnp.where(active, iota8, iota8 + e)).astype(jnp.int32)
    nact = jnp.sum(active.astype(jnp.int32)).reshape(1)

    ib = 1024
    nib = i_dim // ib
    xb = x.astype(jnp.bfloat16)

    def wspec_in(ei, ii, eo, na):
        act = ei < na[0]
        eix = eo[jnp.where(act, ei, na[0] - 1)]
        iix = jnp.where(act, ii, nib - 1)
        return eix, iix

    grid_spec = pltpu.PrefetchScalarGridSpec(
        num_scalar_prefetch=2,
        grid=(e, nib),
        in_specs=[
            pl.BlockSpec((n, h), lambda ei, ii, eo, na: (0, 0)),
            pl.BlockSpec((n, e), lambda ei, ii, eo, na: (0, 0)),
            pl.BlockSpec((1, ib, h),
                         lambda ei, ii, eo, na: (*wspec_in(ei, ii, eo, na), 0)),
            pl.BlockSpec((1, 1, 1, ib),
                         lambda ei, ii, eo, na: (*wspec_in(ei, ii, eo, na), 0, 0)),
            pl.BlockSpec((1, ib, h),
                         lambda ei, ii, eo, na: (*wspec_in(ei, ii, eo, na), 0)),
            pl.BlockSpec((1, 1, 1, ib),
                         lambda ei, ii, eo, na: (*wspec_in(ei, ii, eo, na), 0, 0)),
            pl.BlockSpec((1, h, ib),
                         lambda ei, ii, eo, na:
                         (wspec_in(ei, ii, eo, na)[0], 0,
                          wspec_in(ei, ii, eo, na)[1])),
            pl.BlockSpec((1, 1, h),
                         lambda ei, ii, eo, na:
                         (wspec_in(ei, ii, eo, na)[0], 0, 0)),
        ],
        out_specs=pl.BlockSpec((n, h), lambda ei, ii, eo, na: (0, 0)),
        scratch_shapes=[pltpu.VMEM((n, 1), jnp.float32)],
    )

    combined = pl.pallas_call(
        functools.partial(_ffn_body, n=n, e=e, nib=nib),
        grid_spec=grid_spec,
        out_shape=jax.ShapeDtypeStruct((n, h), jnp.float32),
        compiler_params=pltpu.CompilerParams(
            dimension_semantics=("arbitrary", "arbitrary"),
            vmem_limit_bytes=120 * 1024 * 1024),
    )(eorder, nact, xb, wf, gate_W, gate_b.reshape(e, nib, 1, ib), up_W,
      up_b.reshape(e, nib, 1, ib), down_W, down_b.reshape(e, 1, h))

    return combined.reshape(b, s, h), aux


# speedup vs baseline: 1.0111x; 1.0056x over previous
import functools

import jax
import jax.numpy as jnp
from jax import lax
from jax.experimental import pallas as pl
from jax.experimental.pallas import tpu as pltpu
from jax.experimental.pallas import tpu_sc as plsc

AUXW = 0.01
_NC, _NS, _L = 2, 16, 16


def _router_body(x_ref, rw_ref, wf_ref, scol_ref, lt_ref, *, n, e):
    x = x_ref[...]
    rw = rw_ref[...]
    logits = jax.lax.dot_general(x, rw, (((1,), (1,)), ((), ())),
                                 preferred_element_type=jnp.float32)
    m = jnp.max(logits, axis=1, keepdims=True)
    ex = jnp.exp(logits - m)
    p = ex / jnp.sum(ex, axis=1, keepdims=True)
    iota = jax.lax.broadcasted_iota(jnp.int32, (n, e), 1)
    m1 = jnp.max(p, axis=1, keepdims=True)
    i1 = jnp.min(jnp.where(p == m1, iota, e), axis=1, keepdims=True)
    sel1 = iota == i1
    pm = jnp.where(sel1, -1.0, p)
    m2 = jnp.max(pm, axis=1, keepdims=True)
    i2 = jnp.min(jnp.where(pm == m2, iota, e), axis=1, keepdims=True)
    sel2 = iota == i2
    s = m1 + m2
    wf = jnp.where(sel1, m1 / s, 0.0) + jnp.where(sel2, m2 / s, 0.0)
    wf_ref[...] = wf
    scol_ref[...] = jnp.sum(wf, axis=0, keepdims=True)
    logits_t = jax.lax.dot_general(rw, x, (((1,), (1,)), ((), ())),
                                   preferred_element_type=jnp.float32)
    for ei in range(e):
        lt_ref[pl.ds(ei * n, n)] = logits_t[ei]


def _make_aux_sc(n, e):
    nw = _NC * _NS
    tpw = n // nw
    nch = tpw // _L
    mesh = plsc.VectorSubcoreMesh(core_axis_name="c", subcore_axis_name="s")

    @functools.partial(
        pl.kernel, mesh=mesh,
        out_type=jax.ShapeDtypeStruct((nw, _L), jnp.float32),
        compiler_params=pltpu.CompilerParams(needs_layout_passes=False),
        scratch_types=[
            pltpu.VMEM((tpw * e,), jnp.float32),
            pltpu.VMEM((_L,), jnp.float32),
        ],
    )
    def aux_k(lt_hbm, sall_hbm, lv, svv):
        wid = lax.axis_index("s") * _NC + lax.axis_index("c")
        base = wid * tpw
        for ei in range(e):
            pltpu.sync_copy(lt_hbm.at[pl.ds(ei * n + base, tpw)],
                            lv.at[pl.ds(ei * tpw, tpw)])
        lane = lax.broadcasted_iota(jnp.int32, (_L,), 0)
        accs = [jnp.zeros((_L,), jnp.float32) for _ in range(e)]
        for c in range(nch):
            vs = [lv[pl.ds(ei * tpw + c * _L, _L)] for ei in range(e)]
            m = vs[0]
            for v in vs[1:]:
                m = jnp.maximum(m, v)
            exs = [jnp.exp(v - m) for v in vs]
            ssum = exs[0]
            for ex in exs[1:]:
                ssum = ssum + ex
            ps = [ex / ssum for ex in exs]
            m1 = ps[0]
            for p in ps[1:]:
                m1 = jnp.maximum(m1, p)
            i1 = jnp.full((_L,), e, jnp.int32)
            for ei in range(e):
                i1 = jnp.minimum(i1, jnp.where(
                    ps[ei] == m1, jnp.full((_L,), ei, jnp.int32),
                    jnp.full((_L,), e, jnp.int32)))
            pm = [jnp.where(jnp.full((_L,), ei, jnp.int32) == i1,
                            jnp.full((_L,), -1.0, jnp.float32), ps[ei])
                  for ei in range(e)]
            m2 = pm[0]
            for p in pm[1:]:
                m2 = jnp.maximum(m2, p)
            i2 = jnp.full((_L,), e, jnp.int32)
            for ei in range(e):
                i2 = jnp.minimum(i2, jnp.where(
                    pm[ei] == m2, jnp.full((_L,), ei, jnp.int32),
                    jnp.full((_L,), e, jnp.int32)))
            denom = m1 + m2
            w1 = m1 / denom
            w2 = m2 / denom
            for ei in range(e):
                eivec = jnp.full((_L,), ei, jnp.int32)
                wfe = jnp.where(eivec == i1, w1,
                                jnp.where(eivec == i2, w2,
                                          jnp.zeros((_L,), jnp.float32)))
                accs[ei] = accs[ei] + wfe
        sv = jnp.zeros((_L,), jnp.float32)
        for ei in range(e):
            sv = jnp.where(lane == ei, jnp.sum(accs[ei]), sv)
        svv[...] = sv
        pltpu.sync_copy(svv, sall_hbm.at[wid])

    return aux_k


def _ffn_body(eo_ref, na_ref, x_ref, wf_ref, gw_ref, gb_ref, uw_ref, ub_ref,
              dw_ref, db_ref, out_ref, wcol_ref, *, n, e, nib):
    ei = pl.program_id(0)
    ii = pl.program_id(1)

    @pl.when((ei == 0) & (ii == 0))
    def _():
        out_ref[...] = jnp.zeros_like(out_ref)

    @pl.when(ei < na_ref[0])
    def _():
        emap = eo_ref[jnp.minimum(ei, na_ref[0] - 1)]

        @pl.when(ii == 0)
        def _():
            onehot = (jax.lax.broadcasted_iota(jnp.int32, (e, 1), 0)
                      == emap).astype(jnp.float32)
            wcol_ref[...] = jax.lax.dot_general(
                wf_ref[...], onehot, (((1,), (0,)), ((), ())),
                preferred_element_type=jnp.float32)
            out_ref[...] = out_ref[...] + wcol_ref[...] * db_ref[0]

        wcol = wcol_ref[...]
        x = x_ref[...]
        gw = gw_ref[0].astype(jnp.bfloat16)
        uw = uw_ref[0].astype(jnp.bfloat16)
        dw = dw_ref[0].astype(jnp.bfloat16)
        g = jax.lax.dot_general(x, gw, (((1,), (1,)), ((), ())),
                                preferred_element_type=jnp.float32)
        g = g + gb_ref[0, 0]
        u = jax.lax.dot_general(x, uw, (((1,), (1,)), ((), ())),
                                preferred_element_type=jnp.float32)
        u = u + ub_ref[0, 0]
        a = (g * jax.nn.sigmoid(g) * (u * wcol)).astype(jnp.bfloat16)
        part = jax.lax.dot_general(a, dw, (((1,), (1,)), ((), ())),
                                   preferred_element_type=jnp.float32)
        out_ref[...] = out_ref[...] + part


def kernel(hidden_states, router_W, gate_W, gate_b, up_W, up_b, down_W,
           down_b):
    b, s, h = hidden_states.shape
    e, i_dim = gate_W.shape[:2]
    n = b * s
    x = hidden_states.reshape(n, h)

    wf, scol, lt = pl.pallas_call(
        functools.partial(_router_body, n=n, e=e),
        out_shape=(
            jax.ShapeDtypeStruct((n, e), jnp.float32),
            jax.ShapeDtypeStruct((1, e), jnp.float32),
            jax.ShapeDtypeStruct((n * e,), jnp.float32),
        ),
    )(x, router_W)

    sall = _make_aux_sc(n, e)(lt)
    aux_col = jnp.sum(sall[:, :e], axis=0)
    aux = jnp.sum(aux_col * aux_col) * (AUXW / n)

    iota8 = jnp.arange(e, dtype=jnp.int32)
    active = scol[0] > 0.0
    eorder = jnp.argsort(jnp.where(active, iota8, iota8 + e)).astype(jnp.int32)
    nact = jnp.sum(active.astype(jnp.int32)).reshape(1)

    ib = 1024
    nib = i_dim // ib
    xb = x.astype(jnp.bfloat16)

    def wspec_in(ei, ii, eo, na):
        act = ei < na[0]
        eix = eo[jnp.where(act, ei, na[0] - 1)]
        iix = jnp.where(act, ii, nib - 1)
        return eix, iix

    grid_spec = pltpu.PrefetchScalarGridSpec(
        num_scalar_prefetch=2,
        grid=(e, nib),
        in_specs=[
            pl.BlockSpec((n, h), lambda ei, ii, eo, na: (0, 0)),
            pl.BlockSpec((n, e), lambda ei, ii, eo, na: (0, 0)),
            pl.BlockSpec((1, ib, h),
                         lambda ei, ii, eo, na: (*wspec_in(ei, ii, eo, na), 0)),
            pl.BlockSpec((1, 1, 1, ib),
                         lambda ei, ii, eo, na: (*wspec_in(ei, ii, eo, na), 0, 0)),
            pl.BlockSpec((1, ib, h),
                         lambda ei, ii, eo, na: (*wspec_in(ei, ii, eo, na), 0)),
            pl.BlockSpec((1, 1, 1, ib),
                         lambda ei, ii, eo, na: (*wspec_in(ei, ii, eo, na), 0, 0)),
            pl.BlockSpec((1, h, ib),
                         lambda ei, ii, eo, na:
                         (wspec_in(ei, ii, eo, na)[0], 0,
                          wspec_in(ei, ii, eo, na)[1])),
            pl.BlockSpec((1, 1, h),
                         lambda ei, ii, eo, na:
                         (wspec_in(ei, ii, eo, na)[0], 0, 0)),
        ],
        out_specs=pl.BlockSpec((n, h), lambda ei, ii, eo, na: (0, 0)),
        scratch_shapes=[pltpu.VMEM((n, 1), jnp.float32)],
    )

    combined = pl.pallas_call(
        functools.partial(_ffn_body, n=n, e=e, nib=nib),
        grid_spec=grid_spec,
        out_shape=jax.ShapeDtypeStruct((n, h), jnp.float32),
        compiler_params=pltpu.CompilerParams(
            dimension_semantics=("arbitrary", "arbitrary"),
            vmem_limit_bytes=120 * 1024 * 1024),
    )(eorder, nact, xb, wf, gate_W, gate_b.reshape(e, nib, 1, ib), up_W,
      up_b.reshape(e, nib, 1, ib), down_W, down_b.reshape(e, 1, h))

    return combined.reshape(b, s, h), aux


# speedup vs baseline: 1.0304x; 1.0191x over previous
import functools

import jax
import jax.numpy as jnp
from jax import lax
from jax.experimental import pallas as pl
from jax.experimental.pallas import tpu as pltpu
from jax.experimental.pallas import tpu_sc as plsc

AUXW = 0.01
_NC, _NS, _L = 2, 16, 16


def _router_body(x_ref, rw_ref, wf_ref, scol_ref, lt_ref, xb_ref, *, n, e):
    x = x_ref[...]
    rw = rw_ref[...]
    logits = jax.lax.dot_general(x, rw, (((1,), (1,)), ((), ())),
                                 preferred_element_type=jnp.float32)
    m = jnp.max(logits, axis=1, keepdims=True)
    ex = jnp.exp(logits - m)
    p = ex / jnp.sum(ex, axis=1, keepdims=True)
    iota = jax.lax.broadcasted_iota(jnp.int32, (n, e), 1)
    m1 = jnp.max(p, axis=1, keepdims=True)
    i1 = jnp.min(jnp.where(p == m1, iota, e), axis=1, keepdims=True)
    sel1 = iota == i1
    pm = jnp.where(sel1, -1.0, p)
    m2 = jnp.max(pm, axis=1, keepdims=True)
    i2 = jnp.min(jnp.where(pm == m2, iota, e), axis=1, keepdims=True)
    sel2 = iota == i2
    s = m1 + m2
    wf = jnp.where(sel1, m1 / s, 0.0) + jnp.where(sel2, m2 / s, 0.0)
    wf_ref[...] = wf
    scol_ref[...] = jnp.sum(wf, axis=0, keepdims=True)
    logits_t = jax.lax.dot_general(rw, x, (((1,), (1,)), ((), ())),
                                   preferred_element_type=jnp.float32)
    for ei in range(e):
        lt_ref[pl.ds(ei * n, n)] = logits_t[ei]
    xb_ref[...] = x.astype(jnp.bfloat16)


def _make_aux_sc(n, e):
    nw = _NC * _NS
    tpw = n // nw
    nch = tpw // _L
    mesh = plsc.VectorSubcoreMesh(core_axis_name="c", subcore_axis_name="s")

    @functools.partial(
        pl.kernel, mesh=mesh,
        out_type=jax.ShapeDtypeStruct((nw, _L), jnp.float32),
        compiler_params=pltpu.CompilerParams(needs_layout_passes=False),
        scratch_types=[
            pltpu.VMEM((tpw * e,), jnp.float32),
            pltpu.VMEM((_L,), jnp.float32),
        ],
    )
    def aux_k(lt_hbm, sall_hbm, lv, svv):
        wid = lax.axis_index("s") * _NC + lax.axis_index("c")
        base = wid * tpw
        for ei in range(e):
            pltpu.sync_copy(lt_hbm.at[pl.ds(ei * n + base, tpw)],
                            lv.at[pl.ds(ei * tpw, tpw)])
        lane = lax.broadcasted_iota(jnp.int32, (_L,), 0)
        accs = [jnp.zeros((_L,), jnp.float32) for _ in range(e)]
        for c in range(nch):
            vs = [lv[pl.ds(ei * tpw + c * _L, _L)] for ei in range(e)]
            m = vs[0]
            for v in vs[1:]:
                m = jnp.maximum(m, v)
            exs = [jnp.exp(v - m) for v in vs]
            ssum = exs[0]
            for ex in exs[1:]:
                ssum = ssum + ex
            ps = [ex / ssum for ex in exs]
            m1 = ps[0]
            for p in ps[1:]:
                m1 = jnp.maximum(m1, p)
            i1 = jnp.full((_L,), e, jnp.int32)
            for ei in range(e):
                i1 = jnp.minimum(i1, jnp.where(
                    ps[ei] == m1, jnp.full((_L,), ei, jnp.int32),
                    jnp.full((_L,), e, jnp.int32)))
            pm = [jnp.where(jnp.full((_L,), ei, jnp.int32) == i1,
                            jnp.full((_L,), -1.0, jnp.float32), ps[ei])
                  for ei in range(e)]
            m2 = pm[0]
            for p in pm[1:]:
                m2 = jnp.maximum(m2, p)
            i2 = jnp.full((_L,), e, jnp.int32)
            for ei in range(e):
                i2 = jnp.minimum(i2, jnp.where(
                    pm[ei] == m2, jnp.full((_L,), ei, jnp.int32),
                    jnp.full((_L,), e, jnp.int32)))
            denom = m1 + m2
            w1 = m1 / denom
            w2 = m2 / denom
            for ei in range(e):
                eivec = jnp.full((_L,), ei, jnp.int32)
                wfe = jnp.where(eivec == i1, w1,
                                jnp.where(eivec == i2, w2,
                                          jnp.zeros((_L,), jnp.float32)))
                accs[ei] = accs[ei] + wfe
        sv = jnp.zeros((_L,), jnp.float32)
        for ei in range(e):
            sv = jnp.where(lane == ei, jnp.sum(accs[ei]), sv)
        svv[...] = sv
        pltpu.sync_copy(svv, sall_hbm.at[wid])

    return aux_k


def _ffn_body(eo_ref, na_ref, x_ref, wf_ref, gw_ref, gb_ref, uw_ref, ub_ref,
              dw_ref, db_ref, out_ref, wcol_ref, *, n, e, nib):
    ei = pl.program_id(0)
    ii = pl.program_id(1)

    @pl.when((ei == 0) & (ii == 0))
    def _():
        out_ref[...] = jnp.zeros_like(out_ref)

    @pl.when(ei < na_ref[0])
    def _():
        emap = eo_ref[jnp.minimum(ei, na_ref[0] - 1)]

        @pl.when(ii == 0)
        def _():
            onehot = (jax.lax.broadcasted_iota(jnp.int32, (e, 1), 0)
                      == emap).astype(jnp.float32)
            wcol_ref[...] = jax.lax.dot_general(
                wf_ref[...], onehot, (((1,), (0,)), ((), ())),
                preferred_element_type=jnp.float32)
            out_ref[...] = out_ref[...] + wcol_ref[...] * db_ref[0]

        wcol = wcol_ref[...]
        x = x_ref[...]
        gw = gw_ref[0].astype(jnp.bfloat16)
        uw = uw_ref[0].astype(jnp.bfloat16)
        dw = dw_ref[0].astype(jnp.bfloat16)
        g = jax.lax.dot_general(x, gw, (((1,), (1,)), ((), ())),
                                preferred_element_type=jnp.float32)
        g = g + gb_ref[0, 0]
        u = jax.lax.dot_general(x, uw, (((1,), (1,)), ((), ())),
                                preferred_element_type=jnp.float32)
        u = u + ub_ref[0, 0]
        a = (g * jax.nn.sigmoid(g) * (u * wcol)).astype(jnp.bfloat16)
        part = jax.lax.dot_general(a, dw, (((1,), (1,)), ((), ())),
                                   preferred_element_type=jnp.float32)
        out_ref[...] = out_ref[...] + part


def kernel(hidden_states, router_W, gate_W, gate_b, up_W, up_b, down_W,
           down_b):
    b, s, h = hidden_states.shape
    e, i_dim = gate_W.shape[:2]
    n = b * s
    x = hidden_states.reshape(n, h)

    wf, scol, lt, xb = pl.pallas_call(
        functools.partial(_router_body, n=n, e=e),
        out_shape=(
            jax.ShapeDtypeStruct((n, e), jnp.float32),
            jax.ShapeDtypeStruct((1, e), jnp.float32),
            jax.ShapeDtypeStruct((n * e,), jnp.float32),
            jax.ShapeDtypeStruct((n, h), jnp.bfloat16),
        ),
    )(x, router_W)

    sall = _make_aux_sc(n, e)(lt)
    aux_col = jnp.sum(sall[:, :e], axis=0)
    aux = jnp.sum(aux_col * aux_col) * (AUXW / n)

    iota8 = jnp.arange(e, dtype=jnp.int32)
    active = scol[0] > 0.0
    eorder = jnp.argsort(jnp.where(active, iota8, iota8 + e)).astype(jnp.int32)
    nact = jnp.sum(active.astype(jnp.int32)).reshape(1)

    ib = 1024
    nib = i_dim // ib

    def wspec_in(ei, ii, eo, na):
        act = ei < na[0]
        eix = eo[jnp.where(act, ei, na[0] - 1)]
        iix = jnp.where(act, ii, nib - 1)
        return eix, iix

    grid_spec = pltpu.PrefetchScalarGridSpec(
        num_scalar_prefetch=2,
        grid=(e, nib),
        in_specs=[
            pl.BlockSpec((n, h), lambda ei, ii, eo, na: (0, 0)),
            pl.BlockSpec((n, e), lambda ei, ii, eo, na: (0, 0)),
            pl.BlockSpec((1, ib, h),
                         lambda ei, ii, eo, na: (*wspec_in(ei, ii, eo, na), 0)),
            pl.BlockSpec((1, 1, 1, ib),
                         lambda ei, ii, eo, na: (*wspec_in(ei, ii, eo, na), 0, 0)),
            pl.BlockSpec((1, ib, h),
                         lambda ei, ii, eo, na: (*wspec_in(ei, ii, eo, na), 0)),
            pl.BlockSpec((1, 1, 1, ib),
                         lambda ei, ii, eo, na: (*wspec_in(ei, ii, eo, na), 0, 0)),
            pl.BlockSpec((1, h, ib),
                         lambda ei, ii, eo, na:
                         (wspec_in(ei, ii, eo, na)[0], 0,
                          wspec_in(ei, ii, eo, na)[1])),
            pl.BlockSpec((1, 1, h),
                         lambda ei, ii, eo, na:
                         (wspec_in(ei, ii, eo, na)[0], 0, 0)),
        ],
        out_specs=pl.BlockSpec((n, h), lambda ei, ii, eo, na: (0, 0)),
        scratch_shapes=[pltpu.VMEM((n, 1), jnp.float32)],
    )

    combined = pl.pallas_call(
        functools.partial(_ffn_body, n=n, e=e, nib=nib),
        grid_spec=grid_spec,
        out_shape=jax.ShapeDtypeStruct((n, h), jnp.float32),
        compiler_params=pltpu.CompilerParams(
            dimension_semantics=("arbitrary", "arbitrary"),
            vmem_limit_bytes=120 * 1024 * 1024),
    )(eorder, nact, xb, wf, gate_W, gate_b.reshape(e, nib, 1, ib), up_W,
      up_b.reshape(e, nib, 1, ib), down_W, down_b.reshape(e, 1, h))

    return combined.reshape(b, s, h), aux


# speedup vs baseline: 1.0446x; 1.0137x over previous
import functools

import jax
import jax.numpy as jnp
from jax import lax
from jax.experimental import pallas as pl
from jax.experimental.pallas import tpu as pltpu
from jax.experimental.pallas import tpu_sc as plsc

AUXW = 0.01
_NC, _NS, _L = 2, 16, 16


def _router_body(x_ref, rw_ref, wft_ref, scol_ref, wtf_ref, xb_ref, *, n, e):
    x = x_ref[...]
    rw = rw_ref[...]
    lt = jax.lax.dot_general(rw, x, (((1,), (1,)), ((), ())),
                             preferred_element_type=jnp.float32)
    m = jnp.max(lt, axis=0, keepdims=True)
    ex = jnp.exp(lt - m)
    p = ex / jnp.sum(ex, axis=0, keepdims=True)
    iota = jax.lax.broadcasted_iota(jnp.int32, (e, n), 0)
    m1 = jnp.max(p, axis=0, keepdims=True)
    i1 = jnp.min(jnp.where(p == m1, iota, e), axis=0, keepdims=True)
    sel1 = iota == i1
    pm = jnp.where(sel1, -1.0, p)
    m2 = jnp.max(pm, axis=0, keepdims=True)
    i2 = jnp.min(jnp.where(pm == m2, iota, e), axis=0, keepdims=True)
    sel2 = iota == i2
    s = m1 + m2
    wft = jnp.where(sel1, m1 / s, 0.0) + jnp.where(sel2, m2 / s, 0.0)
    wft_ref[...] = wft
    scol_ref[...] = jnp.sum(wft, axis=1, keepdims=True)
    for ei in range(e):
        wtf_ref[pl.ds(ei * n, n)] = wft[ei]
    xb_ref[...] = x.astype(jnp.bfloat16)


def _make_aux_sc(n, e):
    nw = _NC * _NS
    tpw = n // nw
    nch = tpw // _L
    mesh = plsc.VectorSubcoreMesh(core_axis_name="c", subcore_axis_name="s")

    @functools.partial(
        pl.kernel, mesh=mesh,
        out_type=jax.ShapeDtypeStruct((nw, _L), jnp.float32),
        compiler_params=pltpu.CompilerParams(needs_layout_passes=False),
        scratch_types=[
            pltpu.VMEM((tpw * e,), jnp.float32),
            pltpu.VMEM((_L,), jnp.float32),
        ],
    )
    def aux_k(wtf_hbm, sall_hbm, lv, svv):
        wid = lax.axis_index("s") * _NC + lax.axis_index("c")
        base = wid * tpw
        for ei in range(e):
            pltpu.sync_copy(wtf_hbm.at[pl.ds(ei * n + base, tpw)],
                            lv.at[pl.ds(ei * tpw, tpw)])
        lane = lax.broadcasted_iota(jnp.int32, (_L,), 0)
        sv = jnp.zeros((_L,), jnp.float32)
        for ei in range(e):
            acc = jnp.zeros((_L,), jnp.float32)
            for c in range(nch):
                acc = acc + lv[pl.ds(ei * tpw + c * _L, _L)]
            sv = jnp.where(lane == ei, jnp.sum(acc), sv)
        svv[...] = sv
        pltpu.sync_copy(svv, sall_hbm.at[wid])

    return aux_k


def _ffn_body(eo_ref, na_ref, x_ref, wf_ref, gw_ref, gb_ref, uw_ref, ub_ref,
              dw_ref, db_ref, out_ref, wcol_ref, *, n, e, nib):
    ei = pl.program_id(0)
    ii = pl.program_id(1)

    @pl.when((ei == 0) & (ii == 0))
    def _():
        out_ref[...] = jnp.zeros_like(out_ref)

    @pl.when(ei < na_ref[0])
    def _():
        emap = eo_ref[jnp.minimum(ei, na_ref[0] - 1)]

        @pl.when(ii == 0)
        def _():
            onehot = (jax.lax.broadcasted_iota(jnp.int32, (e, 1), 0)
                      == emap).astype(jnp.float32)
            wcol_ref[...] = jax.lax.dot_general(
                wf_ref[...], onehot, (((0,), (0,)), ((), ())),
                preferred_element_type=jnp.float32)
            out_ref[...] = out_ref[...] + wcol_ref[...] * db_ref[0]

        wcol = wcol_ref[...]
        x = x_ref[...]
        gw = gw_ref[0].astype(jnp.bfloat16)
        uw = uw_ref[0].astype(jnp.bfloat16)
        dw = dw_ref[0].astype(jnp.bfloat16)
        g = jax.lax.dot_general(x, gw, (((1,), (1,)), ((), ())),
                                preferred_element_type=jnp.float32)
        g = g + gb_ref[0, 0]
        u = jax.lax.dot_general(x, uw, (((1,), (1,)), ((), ())),
                                preferred_element_type=jnp.float32)
        u = u + ub_ref[0, 0]
        a = (g * jax.nn.sigmoid(g) * (u * wcol)).astype(jnp.bfloat16)
        part = jax.lax.dot_general(a, dw, (((1,), (1,)), ((), ())),
                                   preferred_element_type=jnp.float32)
        out_ref[...] = out_ref[...] + part


def kernel(hidden_states, router_W, gate_W, gate_b, up_W, up_b, down_W,
           down_b):
    b, s, h = hidden_states.shape
    e, i_dim = gate_W.shape[:2]
    n = b * s
    x = hidden_states.reshape(n, h)

    wft, scol, wtf, xb = pl.pallas_call(
        functools.partial(_router_body, n=n, e=e),
        out_shape=(
            jax.ShapeDtypeStruct((e, n), jnp.float32),
            jax.ShapeDtypeStruct((e, 1), jnp.float32),
            jax.ShapeDtypeStruct((n * e,), jnp.float32),
            jax.ShapeDtypeStruct((n, h), jnp.bfloat16),
        ),
    )(x, router_W)

    sall = _make_aux_sc(n, e)(wtf)
    aux_col = jnp.sum(sall[:, :e], axis=0)
    aux = jnp.sum(aux_col * aux_col) * (AUXW / n)

    iota8 = jnp.arange(e, dtype=jnp.int32)
    active = scol[:, 0] > 0.0
    eorder = jnp.argsort(jnp.where(active, iota8, iota8 + e)).astype(jnp.int32)
    nact = jnp.sum(active.astype(jnp.int32)).reshape(1)

    ib = 1024
    nib = i_dim // ib

    def wspec_in(ei, ii, eo, na):
        act = ei < na[0]
        eix = eo[jnp.where(act, ei, na[0] - 1)]
        iix = jnp.where(act, ii, nib - 1)
        return eix, iix

    grid_spec = pltpu.PrefetchScalarGridSpec(
        num_scalar_prefetch=2,
        grid=(e, nib),
        in_specs=[
            pl.BlockSpec((n, h), lambda ei, ii, eo, na: (0, 0)),
            pl.BlockSpec((e, n), lambda ei, ii, eo, na: (0, 0)),
            pl.BlockSpec((1, ib, h),
                         lambda ei, ii, eo, na: (*wspec_in(ei, ii, eo, na), 0)),
            pl.BlockSpec((1, 1, 1, ib),
                         lambda ei, ii, eo, na: (*wspec_in(ei, ii, eo, na), 0, 0)),
            pl.BlockSpec((1, ib, h),
                         lambda ei, ii, eo, na: (*wspec_in(ei, ii, eo, na), 0)),
            pl.BlockSpec((1, 1, 1, ib),
                         lambda ei, ii, eo, na: (*wspec_in(ei, ii, eo, na), 0, 0)),
            pl.BlockSpec((1, h, ib),
                         lambda ei, ii, eo, na:
                         (wspec_in(ei, ii, eo, na)[0], 0,
                          wspec_in(ei, ii, eo, na)[1])),
            pl.BlockSpec((1, 1, h),
                         lambda ei, ii, eo, na:
                         (wspec_in(ei, ii, eo, na)[0], 0, 0)),
        ],
        out_specs=pl.BlockSpec((n, h), lambda ei, ii, eo, na: (0, 0)),
        scratch_shapes=[pltpu.VMEM((n, 1), jnp.float32)],
    )

    combined = pl.pallas_call(
        functools.partial(_ffn_body, n=n, e=e, nib=nib),
        grid_spec=grid_spec,
        out_shape=jax.ShapeDtypeStruct((n, h), jnp.float32),
        compiler_params=pltpu.CompilerParams(
            dimension_semantics=("arbitrary", "arbitrary"),
            vmem_limit_bytes=120 * 1024 * 1024),
    )(eorder, nact, xb, wft, gate_W, gate_b.reshape(e, nib, 1, ib), up_W,
      up_b.reshape(e, nib, 1, ib), down_W, down_b.reshape(e, 1, h))

    return combined.reshape(b, s, h), aux
